# Initial kernel scaffold; baseline (speedup 1.0000x reference)
import jax
import jax.numpy as jnp
from jax.experimental import pallas as pl


def kernel(table, attention_mask, table_labels_S, table_labels_E, domain_ids, mode, W_S, b_S, W_E, b_E):
    raise NotImplementedError("write your pallas kernel here")


# speedup vs baseline: 2.5589x; 2.5589x over previous
import jax
import jax.numpy as jnp
from jax.experimental import pallas as pl
from jax.experimental.pallas import tpu as pltpu

B, S, D = 4, 96, 768
Z = 0.3
N = B * S * S
RB = 1024
GRID = N // RB


def _proj_body(w_ref, bias_ref, t_ref, o_ref):
    t = t_ref[...].astype(jnp.bfloat16)
    w = w_ref[...].astype(jnp.bfloat16)
    o_ref[...] = (jnp.dot(t, w, preferred_element_type=jnp.float32)
                  + bias_ref[...])


def _select_body(ls_ref, le_ref, vs_ref, ve_ref, am_ref, ms_ref, me_ref):
    mask_len = jnp.sum(am_ref[...], axis=1) - 2
    length = (mask_len.astype(jnp.float32) * Z).astype(jnp.int32)
    length = jnp.maximum(length, 5)
    length = jnp.minimum(length, mask_len * mask_len)
    k = length.reshape(B, 1)

    def masked_pred(l_ref, v_ref):
        valid = (v_ref[...] >= 0).astype(jnp.float32)
        return valid / (1.0 + jnp.exp(-l_ref[...]))

    def search(bits):
        lo = jnp.zeros((B, 1), jnp.int32)
        hi = jnp.full((B, 1), 1 << 30, jnp.int32)

        def it(_, carry):
            lo, hi = carry
            mid = (lo + hi) >> 1
            cnt = jnp.sum((bits >= mid).astype(jnp.int32), axis=1,
                          keepdims=True)
            ge = cnt >= k
            return jnp.where(ge, mid, lo), jnp.where(ge, hi, mid)

        lo, hi = jax.lax.fori_loop(0, 30, it, (lo, hi))
        return lo

    bits_s = jax.lax.bitcast_convert_type(masked_pred(ls_ref, vs_ref),
                                          jnp.int32)
    bits_e = jax.lax.bitcast_convert_type(masked_pred(le_ref, ve_ref),
                                          jnp.int32)
    ms_ref[...] = (bits_s >= search(bits_s)).astype(jnp.int32)
    me_ref[...] = (bits_e >= search(bits_e)).astype(jnp.int32)


def kernel(table, attention_mask, table_labels_S, table_labels_E, domain_ids,
           mode, W_S, b_S, W_E, b_E):
    t2 = table.reshape(N, D)
    w2 = jnp.zeros((D, 128), jnp.float32)
    w2 = w2.at[:, 0].set(W_S[0]).at[:, 1].set(W_E[0])
    bias = jnp.zeros((1, 128), jnp.float32)
    bias = bias.at[0, 0].set(b_S[0]).at[0, 1].set(b_E[0])

    d = pl.pallas_call(
        _proj_body,
        grid=(GRID,),
        in_specs=[
            pl.BlockSpec((D, 128), lambda i: (0, 0)),
            pl.BlockSpec((1, 128), lambda i: (0, 0)),
            pl.BlockSpec((RB, D), lambda i: (i, 0)),
        ],
        out_specs=pl.BlockSpec((RB, 128), lambda i: (i, 0)),
        out_shape=jax.ShapeDtypeStruct((N, 128), jnp.float32),
    )(w2, bias, t2)

    logits_s = d[:, 0].reshape(B, S * S)
    logits_e = d[:, 1].reshape(B, S * S)

    mask_s, mask_e = pl.pallas_call(
        _select_body,
        out_shape=[jax.ShapeDtypeStruct((B, S * S), jnp.int32)] * 2,
    )(logits_s, logits_e,
      table_labels_S.reshape(B, S * S), table_labels_E.reshape(B, S * S),
      attention_mask)

    return (logits_s.reshape(B, S, S), logits_e.reshape(B, S, S),
            mask_s.reshape(B, S, S).astype(bool),
            mask_e.reshape(B, S, S).astype(bool))
